# baseline (speedup 1.0000x reference)
import jax
import jax.numpy as jnp
from jax.experimental import pallas as pl
from jax.experimental.pallas import tpu as pltpu


def _transh_kernel(s1_ref, s2_ref, idx_ref, w_ref, b_ref, hw_ref, re_ref,
                   out1_ref, out2_ref, rel_ref, wr_ref,
                   wbf_ref, hwbf_ref, rebf_ref):
    @pl.when(pl.program_id(1) == 0)
    def _cast_once():
        wbf_ref[...] = w_ref[...].astype(jnp.bfloat16)
        hwbf_ref[...] = hw_ref[...].astype(jnp.bfloat16)
        rebf_ref[...] = re_ref[...].astype(jnp.bfloat16)

    idx = idx_ref[...]
    bt = idx.shape[0]
    r = hw_ref.shape[0]

    iota_r = jax.lax.broadcasted_iota(jnp.int32, (bt, r), 1)
    one_hot = (idx == iota_r).astype(jnp.bfloat16)
    w_r = jnp.dot(one_hot, hwbf_ref[...], preferred_element_type=jnp.float32)
    rel = jnp.dot(one_hot, rebf_ref[...], preferred_element_type=jnp.float32)

    b = b_ref[...]
    h1 = jnp.dot(s1_ref[...].astype(jnp.bfloat16), wbf_ref[...],
                 preferred_element_type=jnp.float32) + b
    h2 = jnp.dot(s2_ref[...].astype(jnp.bfloat16), wbf_ref[...],
                 preferred_element_type=jnp.float32) + b

    out1_ref[...] = h1 - jnp.sum(w_r * h1, axis=-1, keepdims=True) * w_r
    out2_ref[...] = h2 - jnp.sum(w_r * h2, axis=-1, keepdims=True) * w_r
    rel_ref[...] = rel
    wr_ref[...] = w_r


def kernel(sent1_enc, sent2_enc, relation_idx, w_t, b,
           hyperplane_w, relation_embedding):
    B, S = sent1_enc.shape
    M = w_t.shape[1]
    R = hyperplane_w.shape[0]

    idx2d = relation_idx.reshape(B, 1).astype(jnp.int32)
    b2d = b.reshape(1, M)

    bt = min(1024, B)
    n_tiles = pl.cdiv(B, bt)
    n_cores = 2 if n_tiles % 2 == 0 else 1
    nj = n_tiles // n_cores
    grid = (n_cores, nj)

    def tile_map(c, j):
        return (c * nj + j, 0)

    out_shapes = tuple(jax.ShapeDtypeStruct((B, M), jnp.float32)
                       for _ in range(4))
    return pl.pallas_call(
        _transh_kernel,
        out_shape=out_shapes,
        grid=grid,
        in_specs=[
            pl.BlockSpec((bt, S), tile_map),
            pl.BlockSpec((bt, S), tile_map),
            pl.BlockSpec((bt, 1), tile_map),
            pl.BlockSpec((S, M), lambda c, j: (0, 0)),
            pl.BlockSpec((1, M), lambda c, j: (0, 0)),
            pl.BlockSpec((R, M), lambda c, j: (0, 0)),
            pl.BlockSpec((R, M), lambda c, j: (0, 0)),
        ],
        out_specs=(
            pl.BlockSpec((bt, M), tile_map),
            pl.BlockSpec((bt, M), tile_map),
            pl.BlockSpec((bt, M), tile_map),
            pl.BlockSpec((bt, M), tile_map),
        ),
        scratch_shapes=[
            pltpu.VMEM((S, M), jnp.bfloat16),
            pltpu.VMEM((R, M), jnp.bfloat16),
            pltpu.VMEM((R, M), jnp.bfloat16),
        ],
        compiler_params=pltpu.CompilerParams(
            dimension_semantics=("parallel", "arbitrary")),
    )(sent1_enc, sent2_enc, idx2d, w_t, b2d,
      hyperplane_w, relation_embedding)


# speedup vs baseline: 1.0770x; 1.0770x over previous
import jax
import jax.numpy as jnp
from jax.experimental import pallas as pl
from jax.experimental.pallas import tpu as pltpu


def _transh_kernel(s1_ref, s2_ref, idx_ref, w_ref, b_ref, hw_ref, re_ref,
                   out1_ref, out2_ref, rel_ref, wr_ref,
                   wbf_ref, hwbf_ref, rebf_ref):
    @pl.when(pl.program_id(1) == 0)
    def _cast_once():
        wbf_ref[...] = w_ref[...].astype(jnp.bfloat16)
        hwbf_ref[...] = hw_ref[...].astype(jnp.bfloat16)
        rebf_ref[...] = re_ref[...].astype(jnp.bfloat16)

    bt = idx_ref.shape[0]
    idx = idx_ref[...].reshape(bt, 1)
    r = hw_ref.shape[0]

    iota_r = jax.lax.broadcasted_iota(jnp.int32, (bt, r), 1)
    one_hot = (idx == iota_r).astype(jnp.bfloat16)
    w_r = jnp.dot(one_hot, hwbf_ref[...], preferred_element_type=jnp.float32)
    rel = jnp.dot(one_hot, rebf_ref[...], preferred_element_type=jnp.float32)

    b = b_ref[...][None, :]
    h1 = jnp.dot(s1_ref[...].astype(jnp.bfloat16), wbf_ref[...],
                 preferred_element_type=jnp.float32) + b
    h2 = jnp.dot(s2_ref[...].astype(jnp.bfloat16), wbf_ref[...],
                 preferred_element_type=jnp.float32) + b

    out1_ref[...] = h1 - jnp.sum(w_r * h1, axis=-1, keepdims=True) * w_r
    out2_ref[...] = h2 - jnp.sum(w_r * h2, axis=-1, keepdims=True) * w_r
    rel_ref[...] = rel
    wr_ref[...] = w_r


def kernel(sent1_enc, sent2_enc, relation_idx, w_t, b,
           hyperplane_w, relation_embedding):
    B, S = sent1_enc.shape
    M = w_t.shape[1]
    R = hyperplane_w.shape[0]

    bt = min(1024, B)
    n_tiles = pl.cdiv(B, bt)
    n_cores = 2 if n_tiles % 2 == 0 else 1
    nj = n_tiles // n_cores
    grid = (n_cores, nj)

    def tile_map(c, j):
        return (c * nj + j, 0)

    def tile_map1d(c, j):
        return (c * nj + j,)

    out_shapes = tuple(jax.ShapeDtypeStruct((B, M), jnp.float32)
                       for _ in range(4))
    return pl.pallas_call(
        _transh_kernel,
        out_shape=out_shapes,
        grid=grid,
        in_specs=[
            pl.BlockSpec((bt, S), tile_map),
            pl.BlockSpec((bt, S), tile_map),
            pl.BlockSpec((bt,), tile_map1d),
            pl.BlockSpec((S, M), lambda c, j: (0, 0)),
            pl.BlockSpec((M,), lambda c, j: (0,)),
            pl.BlockSpec((R, M), lambda c, j: (0, 0)),
            pl.BlockSpec((R, M), lambda c, j: (0, 0)),
        ],
        out_specs=(
            pl.BlockSpec((bt, M), tile_map),
            pl.BlockSpec((bt, M), tile_map),
            pl.BlockSpec((bt, M), tile_map),
            pl.BlockSpec((bt, M), tile_map),
        ),
        scratch_shapes=[
            pltpu.VMEM((S, M), jnp.bfloat16),
            pltpu.VMEM((R, M), jnp.bfloat16),
            pltpu.VMEM((R, M), jnp.bfloat16),
        ],
        compiler_params=pltpu.CompilerParams(
            dimension_semantics=("parallel", "arbitrary")),
    )(sent1_enc, sent2_enc, relation_idx, w_t, b,
      hyperplane_w, relation_embedding)
